# baseline (speedup 1.0000x reference)
import functools

import jax
import jax.numpy as jnp
from jax import lax
from jax.experimental import pallas as pl
from jax.experimental.pallas import tpu as pltpu
from jax.experimental.pallas import tpu_sc as plsc

B, S, H, V = 4, 2048, 1024, 50257
NC, NS = 2, 16
NW = NC * NS
TOK_PER_W = (B * S) // NW
Q = 32
HQ = Q // 2
NCH = TOK_PER_W // Q
LANES = 16
UNROLL = 8


def _emb_body(x_hbm, tab_hbm, pos_hbm, out_hbm, idx_v, sb_v, pos_v,
              isem0, isem1, psem0, psem1, osem0, osem1):
    wid = lax.axis_index("s") * NC + lax.axis_index("c")
    base = wid * TOK_PER_W
    p0 = lax.rem(base, S)
    isems = (isem0, isem1)
    psems = (psem0, psem1)
    osems = (osem0, osem1)

    pltpu.sync_copy(x_hbm.at[pl.ds(base, TOK_PER_W)], idx_v)

    def gather_desc(c, buf):
        return pltpu.make_async_copy(
            tab_hbm.at[idx_v.at[pl.ds(c * Q, Q)]], sb_v.at[buf], isems[buf])

    def pos_desc(c, half):
        return pltpu.make_async_copy(
            pos_hbm.at[pl.ds(p0 + c * Q + half * HQ, HQ)],
            pos_v.at[half], psems[half])

    def out_desc(c, buf):
        return pltpu.make_async_copy(
            sb_v.at[buf], out_hbm.at[pl.ds(base + c * Q, Q)], osems[buf])

    gather_desc(0, 0).start()
    pos_desc(0, 0).start()
    pos_desc(0, 1).start()

    def add_half(buf, half):
        @plsc.parallel_loop(0, HQ * (H // LANES), unroll=UNROLL)
        def _(k):
            r = lax.shift_right_logical(k, 6)
            off = pl.multiple_of(
                lax.shift_left(lax.bitwise_and(k, 63), 4), LANES)
            sl = pl.ds(off, LANES)
            plsc.addupdate(sb_v.at[buf, half * HQ + r, sl],
                           pos_v[half, r, sl])

    def pair_body(i, _):
        for sub in range(2):
            c = 2 * i + sub
            buf = sub
            obuf = 1 - sub

            @pl.when(c + 1 < NCH)
            def _():
                @pl.when(c >= 1)
                def _():
                    out_desc(c - 1, obuf).wait()

                gather_desc(c + 1, obuf).start()

            gather_desc(c, buf).wait()

            for half in range(2):
                pos_desc(c, half).wait()
                add_half(buf, half)

                @pl.when(c + 1 < NCH)
                def _():
                    pos_desc(c + 1, half).start()

            out_desc(c, buf).start()
        return 0

    lax.fori_loop(0, NCH // 2, pair_body, 0)

    for c in (NCH - 2, NCH - 1):
        out_desc(c, c % 2).wait()


@jax.jit
def _emb(x_flat, table, pos):
    mesh = plsc.VectorSubcoreMesh(core_axis_name="c", subcore_axis_name="s")
    f = functools.partial(
        pl.kernel,
        mesh=mesh,
        out_type=jax.ShapeDtypeStruct((B * S, H), jnp.float32),
        scratch_types=[
            pltpu.VMEM((TOK_PER_W,), jnp.int32),
            pltpu.VMEM((2, Q, H), jnp.float32),
            pltpu.VMEM((2, HQ, H), jnp.float32),
            pltpu.SemaphoreType.DMA,
            pltpu.SemaphoreType.DMA,
            pltpu.SemaphoreType.DMA,
            pltpu.SemaphoreType.DMA,
            pltpu.SemaphoreType.DMA,
            pltpu.SemaphoreType.DMA,
        ],
    )(_emb_body)
    return f(x_flat, table, pos)


def kernel(x, token_table, pos_emb):
    pos = pos_emb.reshape(S, H)
    out = _emb(x.reshape(-1).astype(jnp.int32), token_table, pos)
    return out.reshape(B, S, H)


# speedup vs baseline: 1.1915x; 1.1915x over previous
import functools

import jax
import jax.numpy as jnp
from jax import lax
from jax.experimental import pallas as pl
from jax.experimental.pallas import tpu as pltpu
from jax.experimental.pallas import tpu_sc as plsc

B, S, H, V = 4, 2048, 1024, 50257
NC, NS = 2, 16
NW = NC * NS
SEQ_PER_W = S // NW
P = 8
NCH = SEQ_PER_W // P
LANES = 16
UNROLL = 8


def _emb_body(x_hbm, tab_hbm, pos_hbm, out_hbm, idx_v, sb_v, pos_v,
              isem0, isem1, osem0, osem1):
    wid = lax.axis_index("s") * NC + lax.axis_index("c")
    s0 = wid * SEQ_PER_W
    isems = (isem0, isem1)
    osems = (osem0, osem1)

    for b in range(B):
        pltpu.sync_copy(x_hbm.at[b, pl.ds(s0, SEQ_PER_W)],
                        idx_v.at[pl.ds(b * SEQ_PER_W, SEQ_PER_W)])

    def in_descs(c, buf):
        d = [pltpu.make_async_copy(pos_hbm.at[pl.ds(s0 + c * P, P)],
                                   pos_v.at[buf], isems[buf])]
        for b in range(B):
            d.append(pltpu.make_async_copy(
                tab_hbm.at[idx_v.at[pl.ds(b * SEQ_PER_W + c * P, P)]],
                sb_v.at[buf, b], isems[buf]))
        return d

    def out_descs(c, buf):
        return [pltpu.make_async_copy(
                    sb_v.at[buf],
                    out_hbm.at[:, pl.ds(s0 + c * P, P)], osems[buf])]

    def start(descs):
        for d in descs:
            d.start()

    start(in_descs(0, 0))

    def pair_body(i, _):
        for sub in range(2):
            c = 2 * i + sub
            buf = sub
            obuf = 1 - sub

            @pl.when(c + 1 < NCH)
            def _():
                @pl.when(c >= 1)
                def _():
                    for d in out_descs(c - 1, obuf):
                        d.wait()

                start(in_descs(c + 1, obuf))

            for d in in_descs(c, buf):
                d.wait()

            @plsc.parallel_loop(0, P * (H // LANES), unroll=UNROLL)
            def _(k):
                r = lax.shift_right_logical(k, 6)
                off = pl.multiple_of(
                    lax.shift_left(lax.bitwise_and(k, 63), 4), LANES)
                sl = pl.ds(off, LANES)
                p = pos_v[buf, r, sl]
                for b in range(B):
                    plsc.addupdate(sb_v.at[buf, b, r, sl], p)

            start(out_descs(c, buf))
        return 0

    lax.fori_loop(0, NCH // 2, pair_body, 0)

    for c in (NCH - 2, NCH - 1):
        for d in out_descs(c, c % 2):
            d.wait()


@jax.jit
def _emb(x2d, table, pos):
    mesh = plsc.VectorSubcoreMesh(core_axis_name="c", subcore_axis_name="s")
    f = functools.partial(
        pl.kernel,
        mesh=mesh,
        out_type=jax.ShapeDtypeStruct((B, S, H), jnp.float32),
        scratch_types=[
            pltpu.VMEM((B * SEQ_PER_W,), jnp.int32),
            pltpu.VMEM((2, B, P, H), jnp.float32),
            pltpu.VMEM((2, P, H), jnp.float32),
            pltpu.SemaphoreType.DMA,
            pltpu.SemaphoreType.DMA,
            pltpu.SemaphoreType.DMA,
            pltpu.SemaphoreType.DMA,
        ],
    )(_emb_body)
    return f(x2d, table, pos)


def kernel(x, token_table, pos_emb):
    pos = pos_emb.reshape(S, H)
    return _emb(x.astype(jnp.int32), token_table, pos)


# speedup vs baseline: 1.2100x; 1.0155x over previous
import functools

import jax
import jax.numpy as jnp
from jax import lax
from jax.experimental import pallas as pl
from jax.experimental.pallas import tpu as pltpu
from jax.experimental.pallas import tpu_sc as plsc

B, S, H, V = 4, 2048, 1024, 50257
NC, NS = 2, 16
NW = NC * NS
SEQ_PER_W = S // NW
P = 8
NCH = SEQ_PER_W // P
ROWS = B * P
LANES = 16
UNROLL = 16


def _emb_body(x_hbm, tab_hbm, pos_hbm, out_hbm, idx_v, sb_v, pos_v,
              isem0, isem1, osem0, osem1):
    wid = lax.axis_index("s") * NC + lax.axis_index("c")
    s0 = wid * SEQ_PER_W
    isems = (isem0, isem1)
    osems = (osem0, osem1)

    pltpu.sync_copy(x_hbm.at[wid], idx_v)

    def in_descs(c, buf):
        return [
            pltpu.make_async_copy(pos_hbm.at[pl.ds(s0 + c * P, P)],
                                  pos_v.at[buf], isems[buf]),
            pltpu.make_async_copy(tab_hbm.at[idx_v.at[c]],
                                  sb_v.at[buf], isems[buf]),
        ]

    def out_descs(c, buf):
        return [pltpu.make_async_copy(
                    sb_v.at[buf, pl.ds(b * P, P)],
                    out_hbm.at[b, pl.ds(s0 + c * P, P)], osems[buf])
                for b in range(B)]

    def start(descs):
        for d in descs:
            d.start()

    start(in_descs(0, 0))

    def pair_body(i, _):
        for sub in range(2):
            c = 2 * i + sub
            buf = sub
            obuf = 1 - sub

            @pl.when(c + 1 < NCH)
            def _():
                @pl.when(c >= 1)
                def _():
                    for d in out_descs(c - 1, obuf):
                        d.wait()

                start(in_descs(c + 1, obuf))

            for d in in_descs(c, buf):
                d.wait()

            @plsc.parallel_loop(0, P * (H // LANES), unroll=UNROLL)
            def _(k):
                r = lax.shift_right_logical(k, 6)
                off = pl.multiple_of(
                    lax.shift_left(lax.bitwise_and(k, 63), 4), LANES)
                sl = pl.ds(off, LANES)
                p = pos_v[buf, r, sl]
                for b in range(B):
                    plsc.addupdate(sb_v.at[buf, b * P + r, sl], p)

            start(out_descs(c, buf))
        return 0

    lax.fori_loop(0, NCH // 2, pair_body, 0)

    for c in (NCH - 2, NCH - 1):
        for d in out_descs(c, c % 2):
            d.wait()


@jax.jit
def _emb(x2d, table, pos):
    mesh = plsc.VectorSubcoreMesh(core_axis_name="c", subcore_axis_name="s")
    f = functools.partial(
        pl.kernel,
        mesh=mesh,
        out_type=jax.ShapeDtypeStruct((B, S, H), jnp.float32),
        scratch_types=[
            pltpu.VMEM((NCH, ROWS), jnp.int32),
            pltpu.VMEM((2, ROWS, H), jnp.float32),
            pltpu.VMEM((2, P, H), jnp.float32),
            pltpu.SemaphoreType.DMA,
            pltpu.SemaphoreType.DMA,
            pltpu.SemaphoreType.DMA,
            pltpu.SemaphoreType.DMA,
        ],
    )(_emb_body)
    return f(x2d, table, pos)


def kernel(x, token_table, pos_emb):
    pos = pos_emb.reshape(S, H)
    x2 = (x.astype(jnp.int32)
          .reshape(B, NW, NCH, P)
          .transpose(1, 2, 0, 3)
          .reshape(NW, NCH, ROWS))
    return _emb(x2, token_table, pos)
